# baseline (speedup 1.0000x reference)
import dataclasses
import functools

import jax
import jax.numpy as jnp
from jax import lax
from jax.experimental import pallas as pl
from jax.experimental.pallas import tpu as pltpu
from jax.experimental.pallas import tpu_sc as plsc

_UNITS = 128
_D = 256
_BBL = 128
_UH = 128
_NW = 32
_UC = _UNITS // 16
_SC_ROWS = 1024


def _tc_body(xT_ref, wbc_ref, o_ref):
    m1 = jnp.full((_UH, _BBL), -jnp.inf, dtype=jnp.float32)
    m2 = m1
    for d in range(_D):
        v = wbc_ref[d] + xT_ref[d : d + 1, :]
        t = jnp.minimum(m1, v)
        m1 = jnp.maximum(m1, v)
        m2 = jnp.maximum(m2, t)
    o_ref[...] = (m1 - m2).T


def _tc_top2(x, wt):
    batch = x.shape[0]
    xT = x.T
    wbc = jnp.broadcast_to(wt[:, :, None], (_D, _UNITS, _BBL))
    return pl.pallas_call(
        _tc_body,
        grid=(_UNITS // _UH, batch // _BBL),
        in_specs=[
            pl.BlockSpec((_D, _BBL), lambda uh, i: (0, i)),
            pl.BlockSpec((_D, _UH, _BBL), lambda uh, i: (0, uh, 0)),
        ],
        out_specs=pl.BlockSpec((_BBL, _UH), lambda uh, i: (i, uh)),
        out_shape=jax.ShapeDtypeStruct((batch, _UNITS), jnp.float32),
    )(xT, wbc)


def _sc_top2(x, wt, rows):
    rpw = rows // _NW
    mesh = plsc.VectorSubcoreMesh(core_axis_name="c", subcore_axis_name="s")
    cp = pltpu.CompilerParams()
    if "needs_layout_passes" in pltpu.CompilerParams.__dataclass_fields__:
        cp = dataclasses.replace(cp, needs_layout_passes=False)

    @functools.partial(
        pl.kernel,
        mesh=mesh,
        compiler_params=cp,
        out_type=jax.ShapeDtypeStruct((rows, _UNITS), jnp.float32),
        scratch_types=[
            pltpu.VMEM((rpw, _D), jnp.float32),
            pltpu.VMEM((_D, _UNITS), jnp.float32),
            pltpu.VMEM((rpw, _UNITS), jnp.float32),
        ],
    )
    def sc_kernel(x_hbm, wt_hbm, o_hbm, x_v, wt_v, o_v):
        wid = lax.axis_index("s") * 2 + lax.axis_index("c")
        base = wid * rpw
        pltpu.sync_copy(x_hbm.at[pl.ds(base, rpw)], x_v)
        pltpu.sync_copy(wt_hbm, wt_v)

        @pl.loop(0, rpw)
        def _row(b):
            bfull = jnp.full((16,), b, jnp.int32)
            neg = jnp.full((16,), -jnp.inf, jnp.float32)
            init = (neg,) * (2 * _UC)

            def body(i, carry):
                for half in range(2):
                    d = 2 * i + half
                    m1, m2 = carry[:_UC], carry[_UC:]
                    xs = plsc.load_gather(
                        x_v, [bfull, jnp.full((16,), d, jnp.int32)]
                    )
                    m1n, m2n = [], []
                    for uc in range(_UC):
                        v = xs + wt_v[d, pl.ds(uc * 16, 16)]
                        t = jnp.minimum(m1[uc], v)
                        m1n.append(jnp.maximum(m1[uc], v))
                        m2n.append(jnp.maximum(m2[uc], t))
                    carry = tuple(m1n) + tuple(m2n)
                return carry

            carry = lax.fori_loop(0, _D // 2, body, init)
            for uc in range(_UC):
                o_v[b, pl.ds(uc * 16, 16)] = carry[uc] - carry[_UC + uc]

        pltpu.sync_copy(o_v, o_hbm.at[pl.ds(base, rpw)])

    return sc_kernel(x, wt)


def kernel(inputs, w):
    wt = w.T
    batch = inputs.shape[0]
    if _SC_ROWS <= 0:
        return _tc_top2(inputs, wt)
    if _SC_ROWS >= batch:
        return _sc_top2(inputs, wt, batch)
    sc = _sc_top2(inputs, wt, _SC_ROWS)
    tc = _tc_top2(inputs[_SC_ROWS:], wt)
    return jnp.concatenate([sc, tc], axis=0)


# speedup vs baseline: 1.0526x; 1.0526x over previous
import dataclasses
import functools

import jax
import jax.numpy as jnp
from jax import lax
from jax.experimental import pallas as pl
from jax.experimental.pallas import tpu as pltpu
from jax.experimental.pallas import tpu_sc as plsc

_UNITS = 128
_D = 256
_BBL = 128
_UH = 128
_NW = 32
_UC = _UNITS // 16
_SC_ROWS = 0


def _tc_body(xT_ref, wbc_ref, o_ref):
    m1 = jnp.full((_UH, _BBL), -jnp.inf, dtype=jnp.float32)
    m2 = m1
    for d in range(_D):
        v = wbc_ref[d] + xT_ref[d : d + 1, :]
        t = jnp.minimum(m1, v)
        m1 = jnp.maximum(m1, v)
        m2 = jnp.maximum(m2, t)
    o_ref[...] = (m1 - m2).T


def _tc_top2(x, wt):
    batch = x.shape[0]
    xT = x.T
    wbc = jnp.broadcast_to(wt[:, :, None], (_D, _UNITS, _BBL))
    return pl.pallas_call(
        _tc_body,
        grid=(_UNITS // _UH, batch // _BBL),
        in_specs=[
            pl.BlockSpec((_D, _BBL), lambda uh, i: (0, i)),
            pl.BlockSpec((_D, _UH, _BBL), lambda uh, i: (0, uh, 0)),
        ],
        out_specs=pl.BlockSpec((_BBL, _UH), lambda uh, i: (i, uh)),
        out_shape=jax.ShapeDtypeStruct((batch, _UNITS), jnp.float32),
    )(xT, wbc)


def _sc_top2(x, wt, rows):
    rpw = rows // _NW
    mesh = plsc.VectorSubcoreMesh(core_axis_name="c", subcore_axis_name="s")
    cp = pltpu.CompilerParams()
    if "needs_layout_passes" in pltpu.CompilerParams.__dataclass_fields__:
        cp = dataclasses.replace(cp, needs_layout_passes=False)

    @functools.partial(
        pl.kernel,
        mesh=mesh,
        compiler_params=cp,
        out_type=jax.ShapeDtypeStruct((rows, _UNITS), jnp.float32),
        scratch_types=[
            pltpu.VMEM((rpw, _D), jnp.float32),
            pltpu.VMEM((_D, _UNITS), jnp.float32),
            pltpu.VMEM((rpw, _UNITS), jnp.float32),
        ],
    )
    def sc_kernel(x_hbm, wt_hbm, o_hbm, x_v, wt_v, o_v):
        wid = lax.axis_index("s") * 2 + lax.axis_index("c")
        base = wid * rpw
        pltpu.sync_copy(x_hbm.at[pl.ds(base, rpw)], x_v)
        pltpu.sync_copy(wt_hbm, wt_v)

        @pl.loop(0, rpw)
        def _row(b):
            bfull = jnp.full((16,), b, jnp.int32)
            neg = jnp.full((16,), -jnp.inf, jnp.float32)
            init = (neg,) * (2 * _UC)

            def body(i, carry):
                for half in range(2):
                    d = 2 * i + half
                    m1, m2 = carry[:_UC], carry[_UC:]
                    xs = plsc.load_gather(
                        x_v, [bfull, jnp.full((16,), d, jnp.int32)]
                    )
                    m1n, m2n = [], []
                    for uc in range(_UC):
                        v = xs + wt_v[d, pl.ds(uc * 16, 16)]
                        t = jnp.minimum(m1[uc], v)
                        m1n.append(jnp.maximum(m1[uc], v))
                        m2n.append(jnp.maximum(m2[uc], t))
                    carry = tuple(m1n) + tuple(m2n)
                return carry

            carry = lax.fori_loop(0, _D // 2, body, init)
            for uc in range(_UC):
                o_v[b, pl.ds(uc * 16, 16)] = carry[uc] - carry[_UC + uc]

        pltpu.sync_copy(o_v, o_hbm.at[pl.ds(base, rpw)])

    return sc_kernel(x, wt)


def kernel(inputs, w):
    wt = w.T
    batch = inputs.shape[0]
    if _SC_ROWS <= 0:
        return _tc_top2(inputs, wt)
    if _SC_ROWS >= batch:
        return _sc_top2(inputs, wt, batch)
    sc = _sc_top2(inputs, wt, _SC_ROWS)
    tc = _tc_top2(inputs[_SC_ROWS:], wt)
    return jnp.concatenate([sc, tc], axis=0)
